# baseline (speedup 1.0000x reference)
import functools

import jax
import jax.numpy as jnp
from jax import lax
from jax.experimental import pallas as pl
from jax.experimental.pallas import tpu as pltpu
from jax.experimental.pallas import tpu_sc as plsc


def _transpose2d_kernel(x_ref, o_ref):
    o_ref[...] = x_ref[...].T


def _tc_transpose_table(tt):
    d, v = tt.shape
    bw = 8192
    grid = pl.cdiv(v, bw)
    return pl.pallas_call(
        _transpose2d_kernel,
        grid=(grid,),
        in_specs=[pl.BlockSpec((d, bw), lambda i: (0, i))],
        out_specs=pl.BlockSpec((bw, d), lambda i: (i, 0)),
        out_shape=jax.ShapeDtypeStruct((v, d), jnp.float32),
    )(tt)


def _transpose3d_kernel(x_ref, o_ref):
    o_ref[0] = x_ref[0].T


def _tc_detranspose_out(g3):
    f, b, d = g3.shape
    bb = 2048
    return pl.pallas_call(
        _transpose3d_kernel,
        grid=(f, b // bb),
        in_specs=[pl.BlockSpec((1, bb, d), lambda i, j: (i, j, 0))],
        out_specs=pl.BlockSpec((1, d, bb), lambda i, j: (i, 0, j)),
        out_shape=jax.ShapeDtypeStruct((f, d, b), jnp.float32),
    )(g3)


def _make_sc_gather(b_total: int, d: int):
    info = plsc.get_sparse_core_info()
    nw = info.num_cores * info.num_subcores
    b_per_w = b_total // nw
    chunk = 1024
    n_chunks = b_per_w // chunk

    mesh = plsc.VectorSubcoreMesh(core_axis_name="c", subcore_axis_name="s")

    @functools.partial(
        pl.kernel,
        mesh=mesh,
        out_type=jax.ShapeDtypeStruct((b_total, d), jnp.float32),
        scratch_types=[
            pltpu.VMEM((b_per_w,), jnp.int32),
            pltpu.VMEM((2, chunk, d), jnp.float32),
            pltpu.SemaphoreType.DMA,
            pltpu.SemaphoreType.DMA,
        ],
        compiler_params=pltpu.CompilerParams(use_tc_tiling_on_sc=False),
    )
    def gather_kernel(table_hbm, idx_hbm, out_hbm, idx_v, rows_v, gsem, osem):
        wid = lax.axis_index("s") * info.num_cores + lax.axis_index("c")
        base = wid * b_per_w
        pltpu.sync_copy(idx_hbm.at[pl.ds(base, b_per_w)], idx_v)
        gathers = [None, None]
        outs = [None, None]
        gathers[0] = pltpu.async_copy(
            table_hbm.at[idx_v.at[pl.ds(0, chunk)]], rows_v.at[0], gsem
        )
        for c in range(n_chunks):
            cur = c % 2
            nxt = (c + 1) % 2
            if c + 1 < n_chunks:
                gathers[nxt] = pltpu.async_copy(
                    table_hbm.at[idx_v.at[pl.ds((c + 1) * chunk, chunk)]],
                    rows_v.at[nxt],
                    gsem,
                )
            gathers[cur].wait()
            if outs[cur] is not None:
                outs[cur].wait()
            outs[cur] = pltpu.async_copy(
                rows_v.at[cur],
                out_hbm.at[pl.ds(base + c * chunk, chunk)],
                osem,
            )
        for o in outs:
            if o is not None:
                o.wait()

    return gather_kernel


def kernel(inputs, indices):
    d = inputs.shape[1]
    batch, n_fields = indices.shape
    table_rm = _tc_transpose_table(inputs.T)
    idx_flat = indices.T.reshape(-1)
    gathered = _make_sc_gather(idx_flat.shape[0], d)(table_rm, idx_flat)
    out_t = _tc_detranspose_out(gathered.reshape(n_fields, batch, d))
    return out_t.transpose(2, 0, 1)


# speedup vs baseline: 2.3895x; 2.3895x over previous
import functools

import jax
import jax.numpy as jnp
from jax import lax
from jax.experimental import pallas as pl
from jax.experimental.pallas import tpu as pltpu
from jax.experimental.pallas import tpu_sc as plsc


def _transpose_pad_kernel(x_ref, o_ref):
    y = x_ref[...].T
    o_ref[...] = jnp.concatenate(
        [y, jnp.zeros((y.shape[0], 128 - y.shape[1]), y.dtype)], axis=1
    )


def _tc_transpose_table(tt):
    d, v = tt.shape
    bw = 8192
    grid = pl.cdiv(v, bw)
    return pl.pallas_call(
        _transpose_pad_kernel,
        grid=(grid,),
        in_specs=[pl.BlockSpec((d, bw), lambda i: (0, i))],
        out_specs=pl.BlockSpec((bw, 128), lambda i: (i, 0)),
        out_shape=jax.ShapeDtypeStruct((v, 128), jnp.float32),
    )(tt)


def _slice_transpose_kernel(x_ref, o_ref):
    o_ref[0] = x_ref[0][:, 0:32].T


def _tc_detranspose_out(gp, f, b, d):
    bb = 8192
    return pl.pallas_call(
        _slice_transpose_kernel,
        grid=(f, b // bb),
        in_specs=[pl.BlockSpec((1, bb, 128), lambda i, j: (i, j, 0))],
        out_specs=pl.BlockSpec((1, d, bb), lambda i, j: (i, 0, j)),
        out_shape=jax.ShapeDtypeStruct((f, d, b), jnp.float32),
    )(gp)


def _make_sc_gather(b_total: int, d: int):
    info = plsc.get_sparse_core_info()
    nw = info.num_cores * info.num_subcores
    b_per_w = b_total // nw
    chunk = 1024
    n_chunks = b_per_w // chunk

    mesh = plsc.VectorSubcoreMesh(core_axis_name="c", subcore_axis_name="s")

    @functools.partial(
        pl.kernel,
        mesh=mesh,
        out_type=jax.ShapeDtypeStruct((b_total, 128), jnp.float32),
        scratch_types=[
            pltpu.VMEM((b_per_w,), jnp.int32),
            pltpu.VMEM((2, chunk, d), jnp.float32),
            pltpu.SemaphoreType.DMA,
            pltpu.SemaphoreType.DMA,
        ],
        compiler_params=pltpu.CompilerParams(use_tc_tiling_on_sc=False),
    )
    def gather_kernel(table_hbm, idx_hbm, out_hbm, idx_v, rows_v, gsem, osem):
        wid = lax.axis_index("s") * info.num_cores + lax.axis_index("c")
        base = wid * b_per_w
        pltpu.sync_copy(idx_hbm.at[pl.ds(base, b_per_w)], idx_v)
        gathers = [None, None]
        outs = [None, None]
        gathers[0] = pltpu.async_copy(
            table_hbm.at[idx_v.at[pl.ds(0, chunk)]], rows_v.at[0], gsem
        )
        for c in range(n_chunks):
            cur = c % 2
            nxt = (c + 1) % 2
            if c + 1 < n_chunks:
                gathers[nxt] = pltpu.async_copy(
                    table_hbm.at[idx_v.at[pl.ds((c + 1) * chunk, chunk)]],
                    rows_v.at[nxt],
                    gsem,
                )
            gathers[cur].wait()
            if outs[cur] is not None:
                outs[cur].wait()
            outs[cur] = pltpu.async_copy(
                rows_v.at[cur],
                out_hbm.at[pl.ds(base + c * chunk, chunk), pl.ds(0, d)],
                osem,
            )
        for o in outs:
            if o is not None:
                o.wait()

    return gather_kernel


def kernel(inputs, indices):
    d = inputs.shape[1]
    batch, n_fields = indices.shape
    table_padded = _tc_transpose_table(inputs.T)
    table4 = table_padded.reshape(4 * inputs.shape[0], d)
    idx4 = indices.T.reshape(-1) * 4
    gathered = _make_sc_gather(idx4.shape[0], d)(table4, idx4)
    gp = gathered.reshape(n_fields, batch, 128)
    out_t = _tc_detranspose_out(gp, n_fields, batch, d)
    return out_t.transpose(2, 0, 1)
